# baseline (speedup 1.0000x reference)
import functools

import jax
import jax.numpy as jnp
from jax import lax
from jax.experimental import pallas as pl
from jax.experimental.pallas import tpu as pltpu
from jax.experimental.pallas import tpu_sc as plsc

N_NODES = 10000
D_FEAT = 128
DH = D_FEAT // 2
N_EDGES = 320000

NS = 16
K = 128
NCH = 157
E_PAD = NS * NCH * K
ROWS_PER_TILE = 632


@functools.partial(
    pl.kernel,
    out_type=jax.ShapeDtypeStruct((2, N_NODES, DH), jnp.float32),
    mesh=plsc.VectorSubcoreMesh(core_axis_name="c", subcore_axis_name="s"),
    compiler_params=pltpu.CompilerParams(use_tc_tiling_on_sc=False),
    scratch_types=[
        pltpu.VMEM((NCH, K), jnp.int32),
        pltpu.VMEM((NCH, K), jnp.int32),
        pltpu.VMEM((NCH * K,), jnp.float32),
        pltpu.VMEM((2 * K, DH), jnp.float32),
        pltpu.VMEM_SHARED((N_NODES, DH), jnp.float32),
        pltpu.SemaphoreType.DMA,
        pltpu.SemaphoreType.DMA,
    ],
)
def _spmm_layer(h_hbm, src_hbm, dst_hbm, w_hbm, out_hbm,
                src_v, dst_v, w_v, rows_v, acc, sem_g, sem_s):
    c = lax.axis_index("c")
    s = lax.axis_index("s")

    pltpu.sync_copy(src_hbm.at[s], src_v)
    pltpu.sync_copy(dst_hbm.at[s], dst_v)
    pltpu.sync_copy(w_hbm.at[s], w_v)

    def zrow(i, z):
        for r in range(DH // 16):
            rows_v[i, pl.ds(r * 16, 16)] = jnp.zeros((16,), jnp.float32)
        return z

    lax.fori_loop(0, 2 * K, zrow, 0)
    row_lo = jnp.minimum(s * ROWS_PER_TILE, N_NODES - ROWS_PER_TILE)
    pltpu.sync_copy(rows_v, acc.at[pl.ds(row_lo, 2 * K)])
    pltpu.sync_copy(rows_v, acc.at[pl.ds(row_lo + 2 * K, 2 * K)])
    pltpu.sync_copy(rows_v.at[pl.ds(0, ROWS_PER_TILE - 4 * K)],
                    acc.at[pl.ds(row_lo + 4 * K, ROWS_PER_TILE - 4 * K)])
    plsc.subcore_barrier()

    def buf(p):
        return rows_v.at[pl.ds(p * K, K)]

    pltpu.async_copy(h_hbm.at[c].at[src_v.at[0]], buf(0), sem_g)

    def chunk_body(j, carry):
        p = lax.rem(j, 2)
        pn = 1 - p

        pltpu.make_async_copy(h_hbm.at[c].at[src_v.at[j]], buf(p),
                              sem_g).wait()

        @pl.when(j >= 1)
        def _wait_prev_scatter():
            pltpu.make_async_copy(buf(pn), acc.at[dst_v.at[j - 1]],
                                  sem_s).wait()

        @pl.when(j < NCH - 1)
        def _prefetch_next():
            pltpu.async_copy(h_hbm.at[c].at[src_v.at[j + 1]], buf(pn), sem_g)

        def grp_body(g, c2):
            base = j * K + g * 16
            w16 = w_v[pl.ds(base, 16)]
            for e in range(16):
                wb = jnp.full((16,), w16[e], jnp.float32)
                row = p * K + g * 16 + e
                for r in range(DH // 16):
                    sl = pl.ds(r * 16, 16)
                    rows_v[row, sl] = rows_v[row, sl] * wb
            return c2

        lax.fori_loop(0, K // 16, grp_body, 0)

        pltpu.async_copy(buf(p), acc.at[dst_v.at[j]], sem_s, add=True)
        return carry

    lax.fori_loop(0, NCH, chunk_body, 0)

    pltpu.make_async_copy(buf((NCH - 1) % 2), acc.at[dst_v.at[NCH - 1]],
                          sem_s).wait()
    plsc.subcore_barrier()

    pltpu.sync_copy(acc.at[pl.ds(row_lo, ROWS_PER_TILE)],
                    out_hbm.at[c, pl.ds(row_lo, ROWS_PER_TILE)])


def _fin_body(q_ref, w_ref, b_ref, o_ref):
    h0 = q_ref[0]
    h1 = q_ref[1]
    o_ref[...] = (
        jnp.dot(h0, w_ref[:DH], preferred_element_type=jnp.float32)
        + jnp.dot(h1, w_ref[DH:], preferred_element_type=jnp.float32)
        + b_ref[...]
    )


_tc_finish = pl.pallas_call(
    _fin_body,
    grid=(10,),
    in_specs=[
        pl.BlockSpec((2, N_NODES // 10, DH), lambda i: (0, i, 0)),
        pl.BlockSpec((D_FEAT, D_FEAT), lambda i: (0, 0)),
        pl.BlockSpec((1, D_FEAT), lambda i: (0, 0)),
    ],
    out_specs=pl.BlockSpec((N_NODES // 10, D_FEAT), lambda i: (i, 0)),
    out_shape=jax.ShapeDtypeStruct((N_NODES, D_FEAT), jnp.float32),
)


def kernel(x, edge_index, edge_weight, W, b):
    src = edge_index[0].astype(jnp.int32)
    dst = edge_index[1].astype(jnp.int32)
    w = edge_weight.astype(jnp.float32)

    pad = E_PAD - N_EDGES
    src_p = jnp.concatenate([src, jnp.zeros((pad,), jnp.int32)]).reshape(NS, NCH, K)
    dst_p = jnp.concatenate([dst, jnp.zeros((pad,), jnp.int32)]).reshape(NS, NCH, K)
    w_p = jnp.concatenate([w, jnp.zeros((pad,), jnp.float32)]).reshape(NS, NCH * K)

    x_s = jnp.stack([x[:, :DH], x[:, DH:]])

    P = _spmm_layer(x_s, src_p, dst_p, w_p)
    Q = _spmm_layer(P, src_p, dst_p, w_p)
    return _tc_finish(Q, W.astype(jnp.float32), b.reshape(1, D_FEAT))


# speedup vs baseline: 1.0291x; 1.0291x over previous
import functools

import jax
import jax.numpy as jnp
from jax import lax
from jax.experimental import pallas as pl
from jax.experimental.pallas import tpu as pltpu
from jax.experimental.pallas import tpu_sc as plsc

N_NODES = 10000
D_FEAT = 128
DH = D_FEAT // 2
N_EDGES = 320000

NS = 16
K = 128
NCH = 157
E_PAD = NS * NCH * K
ROWS_PER_TILE = 632


@functools.partial(
    pl.kernel,
    out_type=jax.ShapeDtypeStruct((2, N_NODES, DH), jnp.float32),
    mesh=plsc.VectorSubcoreMesh(core_axis_name="c", subcore_axis_name="s"),
    compiler_params=pltpu.CompilerParams(use_tc_tiling_on_sc=False),
    scratch_types=[
        pltpu.VMEM((3, 2, K), jnp.int32),
        pltpu.VMEM((3, K), jnp.float32),
        pltpu.VMEM((2 * K, DH), jnp.float32),
        pltpu.VMEM_SHARED((N_NODES, DH), jnp.float32),
        pltpu.VMEM_SHARED((N_NODES, DH), jnp.float32),
        pltpu.SemaphoreType.DMA,
        pltpu.SemaphoreType.DMA,
        pltpu.SemaphoreType.DMA,
        pltpu.SemaphoreType.DMA,
    ],
)
def _spmm_layer(h_hbm, idx_hbm, w_hbm, out_hbm,
                idx_v, w_v, rows_v, h_sp, acc, sem_i, sem_w, sem_g, sem_s):
    c = lax.axis_index("c")
    s = lax.axis_index("s")
    row_lo = jnp.minimum(s * ROWS_PER_TILE, N_NODES - ROWS_PER_TILE)

    pltpu.sync_copy(h_hbm.at[c, pl.ds(row_lo, ROWS_PER_TILE)],
                    h_sp.at[pl.ds(row_lo, ROWS_PER_TILE)])

    def zrow(i, z):
        for r in range(DH // 16):
            rows_v[i, pl.ds(r * 16, 16)] = jnp.zeros((16,), jnp.float32)
        return z

    lax.fori_loop(0, 2 * K, zrow, 0)
    pltpu.sync_copy(rows_v, acc.at[pl.ds(row_lo, 2 * K)])
    pltpu.sync_copy(rows_v, acc.at[pl.ds(row_lo + 2 * K, 2 * K)])
    pltpu.sync_copy(rows_v.at[pl.ds(0, ROWS_PER_TILE - 4 * K)],
                    acc.at[pl.ds(row_lo + 4 * K, ROWS_PER_TILE - 4 * K)])
    plsc.subcore_barrier()

    def buf(p):
        return rows_v.at[pl.ds(p * K, K)]

    def idx_copy(j, p):
        return pltpu.make_async_copy(idx_hbm.at[s, j], idx_v.at[p], sem_i)

    def w_copy(j, p):
        return pltpu.make_async_copy(w_hbm.at[s, j], w_v.at[p], sem_w)

    def gather_copy(i3, i2):
        return pltpu.make_async_copy(h_sp.at[idx_v.at[i3, 0]], buf(i2), sem_g)

    def scatter_start(i3, i2):
        pltpu.async_copy(buf(i2), acc.at[idx_v.at[i3, 1]], sem_s, add=True)

    def scatter_wait(i3, i2):
        pltpu.make_async_copy(buf(i2), acc.at[idx_v.at[i3, 1]], sem_s).wait()

    idx_copy(0, 0).start()
    w_copy(0, 0).start()
    idx_copy(0, 0).wait()
    w_copy(0, 0).wait()
    gather_copy(0, 0).start()
    idx_copy(1, 1).start()
    w_copy(1, 1).start()

    def chunk_body(j, carry):
        p3 = lax.rem(j, 3)
        q3 = lax.rem(j + 1, 3)
        r3 = lax.rem(j + 2, 3)
        p2 = lax.rem(j, 2)
        n2 = 1 - p2

        gather_copy(p3, p2).wait()

        @pl.when(j >= 1)
        def _wait_prev_scatter():
            scatter_wait(r3, n2)

        @pl.when(j < NCH - 1)
        def _start_next_gather():
            idx_copy(j + 1, q3).wait()
            w_copy(j + 1, q3).wait()
            gather_copy(q3, n2).start()

        @pl.when(j < NCH - 2)
        def _stage_next_idx():
            idx_copy(j + 2, r3).start()
            w_copy(j + 2, r3).start()

        def grp_body(g, c2):
            w16 = w_v[p3, pl.ds(g * 16, 16)]
            for e in range(16):
                wb = jnp.full((16,), w16[e], jnp.float32)
                row = p2 * K + g * 16 + e
                for r in range(DH // 16):
                    sl = pl.ds(r * 16, 16)
                    rows_v[row, sl] = rows_v[row, sl] * wb
            return c2

        lax.fori_loop(0, K // 16, grp_body, 0)

        scatter_start(p3, p2)
        return carry

    lax.fori_loop(0, NCH, chunk_body, 0)

    scatter_wait((NCH - 1) % 3, (NCH - 1) % 2)
    plsc.subcore_barrier()

    pltpu.sync_copy(acc.at[pl.ds(row_lo, ROWS_PER_TILE)],
                    out_hbm.at[c, pl.ds(row_lo, ROWS_PER_TILE)])


def _fin_body(q_ref, w_ref, b_ref, o_ref):
    h0 = q_ref[0]
    h1 = q_ref[1]
    o_ref[...] = (
        jnp.dot(h0, w_ref[:DH], preferred_element_type=jnp.float32)
        + jnp.dot(h1, w_ref[DH:], preferred_element_type=jnp.float32)
        + b_ref[...]
    )


_tc_finish = pl.pallas_call(
    _fin_body,
    grid=(10,),
    in_specs=[
        pl.BlockSpec((2, N_NODES // 10, DH), lambda i: (0, i, 0)),
        pl.BlockSpec((D_FEAT, D_FEAT), lambda i: (0, 0)),
        pl.BlockSpec((1, D_FEAT), lambda i: (0, 0)),
    ],
    out_specs=pl.BlockSpec((N_NODES // 10, D_FEAT), lambda i: (i, 0)),
    out_shape=jax.ShapeDtypeStruct((N_NODES, D_FEAT), jnp.float32),
)


def kernel(x, edge_index, edge_weight, W, b):
    src = edge_index[0].astype(jnp.int32)
    dst = edge_index[1].astype(jnp.int32)
    w = edge_weight.astype(jnp.float32)

    pad = E_PAD - N_EDGES
    src_p = jnp.concatenate([src, jnp.zeros((pad,), jnp.int32)])
    dst_p = jnp.concatenate([dst, jnp.zeros((pad,), jnp.int32)])
    idx_p = jnp.stack(
        [src_p.reshape(NS, NCH, K), dst_p.reshape(NS, NCH, K)], axis=2)
    w_p = jnp.concatenate([w, jnp.zeros((pad,), jnp.float32)]).reshape(NS, NCH, K)

    x_s = jnp.stack([x[:, :DH], x[:, DH:]])

    P = _spmm_layer(x_s, idx_p, w_p)
    Q = _spmm_layer(P, idx_p, w_p)
    return _tc_finish(Q, W.astype(jnp.float32), b.reshape(1, D_FEAT))


# speedup vs baseline: 2.5466x; 2.4747x over previous
import functools

import jax
import jax.numpy as jnp
from jax import lax
from jax.experimental import pallas as pl
from jax.experimental.pallas import tpu as pltpu
from jax.experimental.pallas import tpu_sc as plsc

N_NODES = 10000
D_FEAT = 128
DH = D_FEAT // 2
N_EDGES = 320000

NS = 16
K = 128
NCH = 157
E_PAD = NS * NCH * K
ROWS_PER_TILE = 632


@functools.partial(
    pl.kernel,
    out_type=jax.ShapeDtypeStruct((2, N_NODES, DH), jnp.float32),
    mesh=plsc.VectorSubcoreMesh(core_axis_name="c", subcore_axis_name="s"),
    compiler_params=pltpu.CompilerParams(use_tc_tiling_on_sc=False),
    scratch_types=[
        pltpu.VMEM((3, 2, K), jnp.int32),
        pltpu.VMEM((3, K), jnp.float32),
        pltpu.VMEM((2 * K, DH), jnp.float32),
        pltpu.VMEM_SHARED((N_NODES, DH), jnp.float32),
        pltpu.VMEM_SHARED((N_NODES, DH), jnp.float32),
        pltpu.SemaphoreType.DMA,
        pltpu.SemaphoreType.DMA,
        pltpu.SemaphoreType.DMA,
        pltpu.SemaphoreType.DMA,
    ],
)
def _spmm_layer(h_hbm, idx_hbm, w_hbm, out_hbm,
                idx_v, w_v, rows_v, h_sp, acc, sem_i, sem_w, sem_g, sem_s):
    c = lax.axis_index("c")
    s = lax.axis_index("s")
    row_lo = jnp.minimum(s * ROWS_PER_TILE, N_NODES - ROWS_PER_TILE)

    pltpu.sync_copy(h_hbm.at[c, pl.ds(row_lo, ROWS_PER_TILE)],
                    h_sp.at[pl.ds(row_lo, ROWS_PER_TILE)])

    def zrow(i, z):
        for r in range(DH // 16):
            rows_v[i, pl.ds(r * 16, 16)] = jnp.zeros((16,), jnp.float32)
        return z

    lax.fori_loop(0, 2 * K, zrow, 0)
    pltpu.sync_copy(rows_v, acc.at[pl.ds(row_lo, 2 * K)])
    pltpu.sync_copy(rows_v, acc.at[pl.ds(row_lo + 2 * K, 2 * K)])
    pltpu.sync_copy(rows_v.at[pl.ds(0, ROWS_PER_TILE - 4 * K)],
                    acc.at[pl.ds(row_lo + 4 * K, ROWS_PER_TILE - 4 * K)])
    plsc.subcore_barrier()

    def buf(p):
        return rows_v.at[pl.ds(p * K, K)]

    def idx_copy(j, p):
        return pltpu.make_async_copy(idx_hbm.at[s, j], idx_v.at[p], sem_i)

    def w_copy(j, p):
        return pltpu.make_async_copy(w_hbm.at[s, j], w_v.at[p], sem_w)

    def gather_copy(i3, i2):
        return pltpu.make_async_copy(h_sp.at[idx_v.at[i3, 0]], buf(i2), sem_g)

    def scatter_start(i3, i2):
        pltpu.async_copy(buf(i2), acc.at[idx_v.at[i3, 1]], sem_s, add=True)

    def scatter_wait(i3, i2):
        pltpu.make_async_copy(buf(i2), acc.at[idx_v.at[i3, 1]], sem_s).wait()

    idx_copy(0, 0).start()
    w_copy(0, 0).start()
    idx_copy(0, 0).wait()
    w_copy(0, 0).wait()
    gather_copy(0, 0).start()
    idx_copy(1, 1).start()
    w_copy(1, 1).start()

    def chunk_body(j, carry):
        p3 = lax.rem(j, 3)
        q3 = lax.rem(j + 1, 3)
        r3 = lax.rem(j + 2, 3)
        p2 = lax.rem(j, 2)
        n2 = 1 - p2

        gather_copy(p3, p2).wait()

        @pl.when(j >= 1)
        def _wait_prev_scatter():
            scatter_wait(r3, n2)

        @pl.when(j < NCH - 1)
        def _start_next_gather():
            idx_copy(j + 1, q3).wait()
            w_copy(j + 1, q3).wait()
            gather_copy(q3, n2).start()

        @pl.when(j < NCH - 2)
        def _stage_next_idx():
            idx_copy(j + 2, r3).start()
            w_copy(j + 2, r3).start()

        row0 = p2 * K
        for g in range(K // 16):
            w16 = w_v[p3, pl.ds(g * 16, 16)]
            for e in range(16):
                wb = jnp.full((16,), w16[e], jnp.float32)
                row = row0 + g * 16 + e
                for r in range(DH // 16):
                    sl = pl.ds(r * 16, 16)
                    rows_v[row, sl] = rows_v[row, sl] * wb

        scatter_start(p3, p2)
        return carry

    lax.fori_loop(0, NCH, chunk_body, 0)

    scatter_wait((NCH - 1) % 3, (NCH - 1) % 2)
    plsc.subcore_barrier()

    pltpu.sync_copy(acc.at[pl.ds(row_lo, ROWS_PER_TILE)],
                    out_hbm.at[c, pl.ds(row_lo, ROWS_PER_TILE)])


def _fin_body(q_ref, w_ref, b_ref, o_ref):
    h0 = q_ref[0]
    h1 = q_ref[1]
    o_ref[...] = (
        jnp.dot(h0, w_ref[:DH], preferred_element_type=jnp.float32)
        + jnp.dot(h1, w_ref[DH:], preferred_element_type=jnp.float32)
        + b_ref[...]
    )


_tc_finish = pl.pallas_call(
    _fin_body,
    grid=(10,),
    in_specs=[
        pl.BlockSpec((2, N_NODES // 10, DH), lambda i: (0, i, 0)),
        pl.BlockSpec((D_FEAT, D_FEAT), lambda i: (0, 0)),
        pl.BlockSpec((1, D_FEAT), lambda i: (0, 0)),
    ],
    out_specs=pl.BlockSpec((N_NODES // 10, D_FEAT), lambda i: (i, 0)),
    out_shape=jax.ShapeDtypeStruct((N_NODES, D_FEAT), jnp.float32),
)


def kernel(x, edge_index, edge_weight, W, b):
    src = edge_index[0].astype(jnp.int32)
    dst = edge_index[1].astype(jnp.int32)
    w = edge_weight.astype(jnp.float32)

    pad = E_PAD - N_EDGES
    src_p = jnp.concatenate([src, jnp.zeros((pad,), jnp.int32)])
    dst_p = jnp.concatenate([dst, jnp.zeros((pad,), jnp.int32)])
    idx_p = jnp.stack(
        [src_p.reshape(NS, NCH, K), dst_p.reshape(NS, NCH, K)], axis=2)
    w_p = jnp.concatenate([w, jnp.zeros((pad,), jnp.float32)]).reshape(NS, NCH, K)

    x_s = jnp.stack([x[:, :DH], x[:, DH:]])

    P = _spmm_layer(x_s, idx_p, w_p)
    Q = _spmm_layer(P, idx_p, w_p)
    return _tc_finish(Q, W.astype(jnp.float32), b.reshape(1, D_FEAT))
